# baseline (speedup 1.0000x reference)
import functools

import jax
import jax.numpy as jnp
from jax import lax
from jax.experimental import pallas as pl
from jax.experimental.pallas import tpu as pltpu
from jax.experimental.pallas import tpu_sc as plsc

TM = 256


def _gelu_exact(x):
    return 0.5 * x * (1.0 + lax.erf(x * 0.7071067811865475))


def _router_body(x_ref, g_ref, b_ref, wr_ref, br_ref,
                 h_ref, gates_ref, idx_ref, *, n_experts):
    xv = x_ref[...]
    mu = jnp.mean(xv, axis=-1, keepdims=True)
    xc = xv - mu
    var = jnp.mean(xc * xc, axis=-1, keepdims=True)
    h = xc * lax.rsqrt(var + 1e-5) * g_ref[0, :] + b_ref[0, :]
    h_ref[...] = h
    logits = jnp.dot(h, wr_ref[...],
                     preferred_element_type=jnp.float32) + br_ref[0, :]
    iota = lax.broadcasted_iota(jnp.int32, logits.shape, 1)
    v1 = jnp.max(logits, axis=-1, keepdims=True)
    i1 = jnp.min(jnp.where(logits >= v1, iota, n_experts),
                 axis=-1, keepdims=True)
    l2 = jnp.where(iota == i1, jnp.float32(-1e30), logits)
    v2 = jnp.max(l2, axis=-1, keepdims=True)
    i2 = jnp.min(jnp.where(l2 >= v2, iota, n_experts),
                 axis=-1, keepdims=True)
    g1 = 1.0 / (1.0 + jnp.exp(v2 - v1))
    col = lax.broadcasted_iota(jnp.int32, (xv.shape[0], 2), 1)
    gates_ref[...] = jnp.where(col == 0, g1, 1.0 - g1)
    idx_ref[...] = jnp.where(col == 0, i1, i2)


def _router(x2, ln_g, ln_b, Wr, br):
    N, D = x2.shape
    E = Wr.shape[1]
    return pl.pallas_call(
        functools.partial(_router_body, n_experts=E),
        out_shape=[
            jax.ShapeDtypeStruct((N, D), jnp.float32),
            jax.ShapeDtypeStruct((N, 2), jnp.float32),
            jax.ShapeDtypeStruct((N, 2), jnp.int32),
        ],
    )(x2, ln_g.reshape(1, D), ln_b.reshape(1, D), Wr, br.reshape(1, E))


def _sc_gather_rows(h, tok, P):
    N, D = h.shape
    info = plsc.get_sparse_core_info()
    NC, NS = info.num_cores, info.num_subcores
    NW = NC * NS
    b_per_w = P // NW
    CH = 80 if b_per_w % 80 == 0 else b_per_w
    n_chunks = b_per_w // CH

    def body(h_hbm, tok_hbm, out_hbm, idx_v, rows_v, sem):
        wid = lax.axis_index("s") * NC + lax.axis_index("c")
        base = wid * b_per_w
        for i in range(n_chunks):
            off = base + i * CH
            pltpu.sync_copy(tok_hbm.at[pl.ds(off, CH)], idx_v)
            pltpu.async_copy(h_hbm.at[idx_v], rows_v, sem).wait()
            pltpu.sync_copy(rows_v, out_hbm.at[pl.ds(off, CH)])

    return pl.kernel(
        body,
        out_type=jax.ShapeDtypeStruct((P, D), jnp.float32),
        mesh=plsc.VectorSubcoreMesh(core_axis_name="c", subcore_axis_name="s"),
        scratch_types=[
            pltpu.VMEM((CH,), jnp.int32),
            pltpu.VMEM((CH, D), jnp.float32),
            pltpu.SemaphoreType.DMA,
        ],
    )(h, tok)


def _ffn_body(te_ref, h_ref, w1_ref, b1_ref, w2_ref, b2_ref, g_ref, z_ref):
    h = h_ref[...].astype(jnp.bfloat16)
    hid = jnp.dot(h, w1_ref[0], preferred_element_type=jnp.float32)
    hid = _gelu_exact(hid + b1_ref[0, 0, :])
    y = jnp.dot(hid.astype(jnp.bfloat16), w2_ref[0],
                preferred_element_type=jnp.float32)
    z_ref[...] = (y + b2_ref[0, 0, :]) * g_ref[...]


def _grouped_ffn(h_sorted, tile_expert, g_sorted, W1b, b1, W2b, b2):
    P, D = h_sorted.shape
    E, _, DFF = W1b.shape
    NT = P // TM
    grid_spec = pltpu.PrefetchScalarGridSpec(
        num_scalar_prefetch=1,
        grid=(NT,),
        in_specs=[
            pl.BlockSpec((TM, D), lambda m, te: (m, 0)),
            pl.BlockSpec((1, D, DFF), lambda m, te: (te[m], 0, 0)),
            pl.BlockSpec((1, 1, DFF), lambda m, te: (te[m], 0, 0)),
            pl.BlockSpec((1, DFF, D), lambda m, te: (te[m], 0, 0)),
            pl.BlockSpec((1, 1, D), lambda m, te: (te[m], 0, 0)),
            pl.BlockSpec((TM, 1), lambda m, te: (m, 0)),
        ],
        out_specs=pl.BlockSpec((TM, D), lambda m, te: (m, 0)),
    )
    return pl.pallas_call(
        _ffn_body,
        grid_spec=grid_spec,
        out_shape=jax.ShapeDtypeStruct((P, D), jnp.float32),
        compiler_params=pltpu.CompilerParams(
            dimension_semantics=("arbitrary",),
            vmem_limit_bytes=100 * 1024 * 1024,
        ),
    )(tile_expert, h_sorted, W1b, b1.reshape(E, 1, DFF), W2b,
      b2.reshape(E, 1, D), g_sorted)


def _sc_combine(x2, z, p0, p1):
    N, D = x2.shape
    info = plsc.get_sparse_core_info()
    NC, NS, L = info.num_cores, info.num_subcores, info.num_lanes
    NW = NC * NS
    b_per_w = N // NW
    CH = 32
    n_chunks = b_per_w // CH
    n_lane_blocks = D // L

    def body(x_hbm, z_hbm, p0_hbm, p1_hbm, out_hbm,
             i0_v, i1_v, xb, z0b, z1b, sem):
        wid = lax.axis_index("s") * NC + lax.axis_index("c")
        base = wid * b_per_w
        for i in range(n_chunks):
            off = base + i * CH
            pltpu.sync_copy(p0_hbm.at[pl.ds(off, CH)], i0_v)
            pltpu.sync_copy(p1_hbm.at[pl.ds(off, CH)], i1_v)
            pltpu.sync_copy(x_hbm.at[pl.ds(off, CH)], xb)
            pltpu.async_copy(z_hbm.at[i0_v], z0b, sem).wait()
            pltpu.async_copy(z_hbm.at[i1_v], z1b, sem).wait()

            def row_add(r, carry):
                for c in range(n_lane_blocks):
                    s = pl.ds(c * L, L)
                    xb[r, s] = xb[r, s] + z0b[r, s] + z1b[r, s]
                return carry

            lax.fori_loop(0, CH, row_add, 0)
            pltpu.sync_copy(xb, out_hbm.at[pl.ds(off, CH)])

    return pl.kernel(
        body,
        out_type=jax.ShapeDtypeStruct((N, D), jnp.float32),
        mesh=plsc.VectorSubcoreMesh(core_axis_name="c", subcore_axis_name="s"),
        scratch_types=[
            pltpu.VMEM((CH,), jnp.int32),
            pltpu.VMEM((CH,), jnp.int32),
            pltpu.VMEM((CH, D), jnp.float32),
            pltpu.VMEM((CH, D), jnp.float32),
            pltpu.VMEM((CH, D), jnp.float32),
            pltpu.SemaphoreType.DMA,
        ],
    )(x2, z, p0, p1)


def kernel(x, ln_g, ln_b, Wr, br, W1, b1, W2, b2):
    B, T, D = x.shape
    E = Wr.shape[1]
    DFF = W1.shape[2]
    N = B * T
    K = 2
    NK = N * K
    P = NK + E * TM

    x2 = x.reshape(N, D)
    h, gates, idx = _router(x2, ln_g, ln_b, Wr, br)

    idx_flat = idx.reshape(NK)
    onehot = (idx_flat[:, None] == jnp.arange(E, dtype=jnp.int32)
              ).astype(jnp.int32)
    counts = jnp.sum(onehot, axis=0)
    ranks = jnp.sum(onehot * (jnp.cumsum(onehot, axis=0) - onehot),
                    axis=1)
    padded = ((counts + TM - 1) // TM) * TM
    off = jnp.concatenate([jnp.zeros((1,), jnp.int32),
                           jnp.cumsum(padded)[:-1].astype(jnp.int32)])
    pos = off[idx_flat] + ranks
    row_tok = jnp.zeros((P,), jnp.int32).at[pos].set(
        jnp.arange(NK, dtype=jnp.int32) // K)
    g_sorted = jnp.zeros((P,), jnp.float32).at[pos].set(
        gates.reshape(NK)).reshape(P, 1)
    bnd = jnp.cumsum(padded // TM)
    t_iota = jnp.arange(P // TM, dtype=jnp.int32)
    tile_expert = jnp.minimum(
        jnp.sum((t_iota[:, None] >= bnd[None, :]).astype(jnp.int32), axis=1),
        E - 1).astype(jnp.int32)
    pos2 = pos.reshape(N, K)

    h_sorted = _sc_gather_rows(h, row_tok, P)
    z = _grouped_ffn(h_sorted, tile_expert, g_sorted,
                     W1.astype(jnp.bfloat16), b1,
                     W2.astype(jnp.bfloat16), b2)
    out = _sc_combine(x2, z, pos2[:, 0], pos2[:, 1])
    return out.reshape(B, T, D)
